# baseline (speedup 1.0000x reference)
import numpy as np

import jax
import jax.numpy as jnp
from jax import lax
from jax.experimental import pallas as pl
from jax.experimental.pallas import tpu as pltpu
from jax.experimental.pallas import tpu_sc as plsc

BATCH = 16384
HIST = 50
D = 64
B_TOT = BATCH * HIST
NW = 32
PER_W = B_TOT // NW
CHUNK = 128
N_CHUNKS = PER_W // CHUNK

_MASK_ME = np.int32(0x0FFFE000)
_MASK_S = np.int32(-2147483648)
_SCALE = np.float32(2.0 ** 112)


def _cvt(me, s):
    f = lax.bitcast_convert_type(me, jnp.float32) * _SCALE
    return lax.bitcast_convert_type(
        s | lax.bitcast_convert_type(f, jnp.int32), jnp.float32)


def _emb_body(tbl_hbm, idx_hbm, out_hbm, idx_v, rows_v, out_v, sem):
    wid = lax.axis_index("s") * 2 + lax.axis_index("c")
    base = wid * PER_W
    lanes2 = lax.iota(jnp.int32, 16) * 2

    def chunk_body(c, carry):
        off = base + c * CHUNK
        pltpu.sync_copy(idx_hbm.at[pl.ds(off, CHUNK)], idx_v)
        pltpu.async_copy(tbl_hbm.at[idx_v], rows_v, sem).wait()

        def row_body(r, carry2):
            rbase = jnp.full((16,), r * D, jnp.int32) + lanes2
            for j in range(2):
                w = rows_v[r, pl.ds(j * 16, 16)]
                r_lo = _cvt((w << 13) & _MASK_ME, (w << 16) & _MASK_S)
                r_hi = _cvt((w >> 3) & _MASK_ME, w & _MASK_S)
                col = rbase + (j * 32)
                plsc.store_scatter(out_v, [col], r_lo)
                plsc.store_scatter(out_v, [col + 1], r_hi)
            return carry2

        lax.fori_loop(0, CHUNK, row_body, 0)
        pltpu.sync_copy(out_v, out_hbm.at[pl.ds(off * D, CHUNK * D)])
        return carry

    lax.fori_loop(0, N_CHUNKS, chunk_body, 0)


_emb = pl.kernel(
    _emb_body,
    out_type=jax.ShapeDtypeStruct((B_TOT * D,), jnp.float32),
    mesh=plsc.VectorSubcoreMesh(core_axis_name="c", subcore_axis_name="s"),
    compiler_params=pltpu.CompilerParams(
        needs_layout_passes=False, use_tc_tiling_on_sc=False),
    scratch_types=[
        pltpu.VMEM((CHUNK,), jnp.int32),
        pltpu.VMEM((CHUNK, D // 2), jnp.int32),
        pltpu.VMEM((CHUNK * D,), jnp.float32),
        pltpu.SemaphoreType.DMA,
    ],
)


@jax.jit
def kernel(x, weight):
    idx = x.reshape(-1)
    tbl = jax.lax.bitcast_convert_type(
        weight.reshape(weight.shape[0], D // 2, 2), jnp.int32)
    out = _emb(tbl, idx)
    return out.reshape(BATCH, HIST, D)


# speedup vs baseline: 1.1359x; 1.1359x over previous
import numpy as np

import jax
import jax.numpy as jnp
from jax import lax
from jax.experimental import pallas as pl
from jax.experimental.pallas import tpu as pltpu
from jax.experimental.pallas import tpu_sc as plsc

BATCH = 16384
HIST = 50
D = 64
DW = D // 2
B_TOT = BATCH * HIST
NW = 32
PER_W = B_TOT // NW
CHUNK = 128
N_CHUNKS = PER_W // CHUNK
NBUF = 2
N_SUPER = N_CHUNKS // NBUF

_MASK_ME = np.int32(0x0FFFE000)
_MASK_S = np.int32(-2147483648)
_SCALE = np.float32(2.0 ** 112)


def _emb_body(tbl_hbm, idx_hbm, out_hbm,
              idx_all, rows0, rows1, out0, out1,
              gsem0, gsem1, osem0, osem1):
    wid = lax.axis_index("s") * 2 + lax.axis_index("c")
    cbase = wid * N_CHUNKS
    base = wid * PER_W
    lanes2 = lax.iota(jnp.int32, 16) * 2

    rows = [rows0, rows1]
    outs = [out0, out1]
    gsems = [gsem0, gsem1]
    osems = [osem0, osem1]

    pltpu.sync_copy(idx_hbm.at[pl.ds(cbase, N_CHUNKS)], idx_all)

    for b in range(NBUF):
        pltpu.async_copy(tbl_hbm.at[idx_all.at[b]], rows[b], gsems[b])

    def super_body(s, carry):
        for b in range(NBUF):
            c = s * NBUF + b
            pltpu.make_async_copy(
                tbl_hbm.at[idx_all.at[c]], rows[b], gsems[b]).wait()

            @pl.when(s > 0)
            def _wait_out():
                pltpu.make_async_copy(
                    outs[b], out_hbm.at[pl.ds(base * D, CHUNK * D)],
                    osems[b]).wait()

            def row_body(r, carry2):
                rbase = jnp.full((16,), r * D, jnp.int32) + lanes2
                for j in range(2):
                    w = rows[b][r, pl.ds(j * 16, 16)]
                    f_lo = lax.bitcast_convert_type(
                        ((w << 13) & _MASK_ME) | ((w << 16) & _MASK_S),
                        jnp.float32) * _SCALE
                    f_hi = lax.bitcast_convert_type(
                        ((w >> 3) & _MASK_ME) | (w & _MASK_S),
                        jnp.float32) * _SCALE
                    col = rbase + (j * 32)
                    plsc.store_scatter(outs[b], [col], f_lo)
                    plsc.store_scatter(outs[b], [col + 1], f_hi)
                return carry2

            lax.fori_loop(0, CHUNK, row_body, 0, unroll=4)

            pltpu.async_copy(
                outs[b], out_hbm.at[pl.ds((base + c * CHUNK) * D, CHUNK * D)],
                osems[b])

            @pl.when(s < N_SUPER - 1)
            def _next_gather():
                pltpu.async_copy(
                    tbl_hbm.at[idx_all.at[c + NBUF]], rows[b], gsems[b])

        return carry

    lax.fori_loop(0, N_SUPER, super_body, 0)

    for b in range(NBUF):
        pltpu.make_async_copy(
            outs[b], out_hbm.at[pl.ds(base * D, CHUNK * D)], osems[b]).wait()


_emb = pl.kernel(
    _emb_body,
    out_type=jax.ShapeDtypeStruct((B_TOT * D,), jnp.float32),
    mesh=plsc.VectorSubcoreMesh(core_axis_name="c", subcore_axis_name="s"),
    compiler_params=pltpu.CompilerParams(
        needs_layout_passes=False, use_tc_tiling_on_sc=False),
    scratch_types=[
        pltpu.VMEM((N_CHUNKS, CHUNK), jnp.int32),
        pltpu.VMEM((CHUNK, DW), jnp.int32),
        pltpu.VMEM((CHUNK, DW), jnp.int32),
        pltpu.VMEM((CHUNK * D,), jnp.float32),
        pltpu.VMEM((CHUNK * D,), jnp.float32),
        pltpu.SemaphoreType.DMA,
        pltpu.SemaphoreType.DMA,
        pltpu.SemaphoreType.DMA,
        pltpu.SemaphoreType.DMA,
    ],
)


@jax.jit
def kernel(x, weight):
    idx = x.reshape(B_TOT // CHUNK, CHUNK)
    tbl = jax.lax.bitcast_convert_type(
        weight.reshape(weight.shape[0], DW, 2), jnp.int32)
    out = _emb(tbl, idx)
    return out.reshape(BATCH, HIST, D)
